# baseline (speedup 1.0000x reference)
import functools

import jax
import jax.numpy as jnp
from jax import lax
from jax.experimental import pallas as pl
from jax.experimental.pallas import tpu as pltpu
from jax.experimental.pallas import tpu_sc as plsc

B = 16384
D = 32
NUM_TABLES = 4


def _build_sc_gather():
    info = plsc.get_sparse_core_info()
    nc, ns = info.num_cores, info.num_subcores
    nw = nc * ns
    bpw = B // nw
    mesh = plsc.VectorSubcoreMesh(core_axis_name="c", subcore_axis_name="s")

    @functools.partial(
        pl.kernel,
        mesh=mesh,
        out_type=tuple(
            jax.ShapeDtypeStruct((B, D), jnp.float32) for _ in range(NUM_TABLES)
        ),
        scratch_types=[
            pltpu.VMEM((bpw,), jnp.int32),
            pltpu.VMEM((bpw, D), jnp.float32),
            pltpu.SemaphoreType.DMA,
        ],
        compiler_params=pltpu.CompilerParams(use_tc_tiling_on_sc=False),
    )
    def gather4(g_idx, a_idx, o_idx, r_idx, w_g, w_a, w_o, w_r,
                out_g, out_a, out_o, out_r, idx_v, rows_v, sem):
        wid = lax.axis_index("s") * nc + lax.axis_index("c")
        base = wid * bpw
        for idx_hbm, tab_hbm, out_hbm in (
            (g_idx, w_g, out_g),
            (a_idx, w_a, out_a),
            (o_idx, w_o, out_o),
            (r_idx, w_r, out_r),
        ):
            pltpu.sync_copy(idx_hbm.at[pl.ds(base, bpw)], idx_v)
            pltpu.async_copy(tab_hbm.at[idx_v], rows_v, sem).wait()
            pltpu.sync_copy(rows_v, out_hbm.at[pl.ds(base, bpw)])

    return gather4


_sc_gather = _build_sc_gather()


def _tc_normalize(g, a, o, r):
    bs = 2048

    def body(g_ref, a_ref, o_ref, r_ref, out_ref):
        for i, ref in enumerate((g_ref, a_ref, o_ref, r_ref)):
            x = ref[...]
            norms = jnp.sqrt(jnp.sum(x * x, axis=1, keepdims=True))
            scale = jnp.minimum(1.0, 1.0 / jnp.maximum(norms, 1e-7))
            out_ref[:, i * D:(i + 1) * D] = x * scale

    return pl.pallas_call(
        body,
        grid=(B // bs,),
        in_specs=[pl.BlockSpec((bs, D), lambda i: (i, 0))] * NUM_TABLES,
        out_specs=pl.BlockSpec((bs, NUM_TABLES * D), lambda i: (i, 0)),
        out_shape=jax.ShapeDtypeStruct((B, NUM_TABLES * D), jnp.float32),
    )(g, a, o, r)


def kernel(gender_idx, age_idx, occupation_idx, area_idx,
           W_gender, W_age, W_occupation, W_area):
    g, a, o, r = _sc_gather(
        gender_idx.astype(jnp.int32),
        age_idx.astype(jnp.int32),
        occupation_idx.astype(jnp.int32),
        area_idx.astype(jnp.int32),
        W_gender, W_age, W_occupation, W_area,
    )
    return _tc_normalize(g, a, o, r)


# speedup vs baseline: 2.1445x; 2.1445x over previous
import functools

import jax
import jax.numpy as jnp
from jax import lax
from jax.experimental import pallas as pl
from jax.experimental.pallas import tpu as pltpu
from jax.experimental.pallas import tpu_sc as plsc

B = 16384
D = 32
NUM_TABLES = 4
SMALL_ROWS = (2, 7, 21)


def _build_sc_gather():
    info = plsc.get_sparse_core_info()
    nc, ns, nl = info.num_cores, info.num_subcores, info.num_lanes
    nw = nc * ns
    bpw = B // nw
    mesh = plsc.VectorSubcoreMesh(core_axis_name="c", subcore_axis_name="s")

    scratch = [
        pltpu.VMEM((bpw,), jnp.int32),
        pltpu.VMEM((bpw, D), jnp.float32),
        pltpu.SemaphoreType.DMA,
    ]
    for v in SMALL_ROWS:
        scratch.append(pltpu.VMEM((bpw,), jnp.int32))
        scratch.append(pltpu.VMEM((v, D), jnp.float32))
        scratch.append(pltpu.VMEM((bpw, D), jnp.float32))

    @functools.partial(
        pl.kernel,
        mesh=mesh,
        out_type=tuple(
            jax.ShapeDtypeStruct((B, D), jnp.float32) for _ in range(NUM_TABLES)
        ),
        scratch_types=scratch,
        compiler_params=pltpu.CompilerParams(
            use_tc_tiling_on_sc=False, needs_layout_passes=False),
    )
    def gather4(g_idx, a_idx, o_idx, r_idx, w_g, w_a, w_o, w_r,
                out_g, out_a, out_o, out_r,
                r_idx_v, r_rows_v, r_sem,
                g_idx_v, g_tab_v, g_rows_v,
                a_idx_v, a_tab_v, a_rows_v,
                o_idx_v, o_tab_v, o_rows_v):
        wid = lax.axis_index("s") * nc + lax.axis_index("c")
        base = wid * bpw

        pltpu.sync_copy(r_idx.at[pl.ds(base, bpw)], r_idx_v)
        area_dma = pltpu.async_copy(w_r.at[r_idx_v], r_rows_v, r_sem)

        small = (
            (g_idx, w_g, out_g, g_idx_v, g_tab_v, g_rows_v),
            (a_idx, w_a, out_a, a_idx_v, a_tab_v, a_rows_v),
            (o_idx, w_o, out_o, o_idx_v, o_tab_v, o_rows_v),
        )
        for idx_hbm, tab_hbm, out_hbm, idx_v, tab_v, rows_v in small:
            pltpu.sync_copy(idx_hbm.at[pl.ds(base, bpw)], idx_v)
            pltpu.sync_copy(tab_hbm, tab_v)

        lane = lax.iota(jnp.int32, nl)
        cols = [jnp.full((nl,), j, jnp.int32) for j in range(D)]

        for idx_hbm, tab_hbm, out_hbm, idx_v, tab_v, rows_v in small:
            def chunk_body(c, _, idx_v=idx_v, tab_v=tab_v, rows_v=rows_v):
                idx16 = idx_v[pl.ds(c * nl, nl)]
                row16 = c * nl + lane
                for j in range(D):
                    v = plsc.load_gather(tab_v, [idx16, cols[j]])
                    plsc.store_scatter(rows_v, [row16, cols[j]], v)
                return 0
            lax.fori_loop(0, bpw // nl, chunk_body, 0)
            pltpu.sync_copy(rows_v, out_hbm.at[pl.ds(base, bpw)])

        area_dma.wait()
        pltpu.sync_copy(r_rows_v, out_r.at[pl.ds(base, bpw)])

    return gather4


_sc_gather = _build_sc_gather()


def _tc_normalize(g, a, o, r):
    bs = 2048

    def body(g_ref, a_ref, o_ref, r_ref, out_ref):
        for i, ref in enumerate((g_ref, a_ref, o_ref, r_ref)):
            x = ref[...]
            norms = jnp.sqrt(jnp.sum(x * x, axis=1, keepdims=True))
            scale = jnp.minimum(1.0, 1.0 / jnp.maximum(norms, 1e-7))
            out_ref[:, i * D:(i + 1) * D] = x * scale

    return pl.pallas_call(
        body,
        grid=(B // bs,),
        in_specs=[pl.BlockSpec((bs, D), lambda i: (i, 0))] * NUM_TABLES,
        out_specs=pl.BlockSpec((bs, NUM_TABLES * D), lambda i: (i, 0)),
        out_shape=jax.ShapeDtypeStruct((B, NUM_TABLES * D), jnp.float32),
    )(g, a, o, r)


def kernel(gender_idx, age_idx, occupation_idx, area_idx,
           W_gender, W_age, W_occupation, W_area):
    g, a, o, r = _sc_gather(
        gender_idx.astype(jnp.int32),
        age_idx.astype(jnp.int32),
        occupation_idx.astype(jnp.int32),
        area_idx.astype(jnp.int32),
        W_gender, W_age, W_occupation, W_area,
    )
    return _tc_normalize(g, a, o, r)


# speedup vs baseline: 2.7584x; 1.2863x over previous
import functools

import jax
import jax.numpy as jnp
from jax import lax
from jax.experimental import pallas as pl
from jax.experimental.pallas import tpu as pltpu
from jax.experimental.pallas import tpu_sc as plsc

B = 16384
D = 32
OUT_D = 128
AREA_FOLD = 4
SMALL_ROWS = (2, 7, 21)


def _rsqrt_nr(s):
    i = plsc.bitcast(s, jnp.int32)
    i = jnp.int32(0x5F3759DF) - jnp.right_shift(i, 1)
    y = plsc.bitcast(i, jnp.float32)
    for _ in range(3):
        y = y * (1.5 - 0.5 * s * y * y)
    return y


def _build_sc_kernel():
    info = plsc.get_sparse_core_info()
    nc, ns, nl = info.num_cores, info.num_subcores, info.num_lanes
    nw = nc * ns
    bpw = B // nw
    half = bpw // 2
    mesh = plsc.VectorSubcoreMesh(core_axis_name="c", subcore_axis_name="s")

    scratch = [
        pltpu.VMEM((bpw,), jnp.int32),
        pltpu.VMEM((bpw,), jnp.int32),
        pltpu.VMEM((half, OUT_D), jnp.float32),
        pltpu.VMEM((half, OUT_D), jnp.float32),
        pltpu.VMEM((half, OUT_D), jnp.float32),
        pltpu.SemaphoreType.DMA,
        pltpu.SemaphoreType.DMA,
    ]
    for v in SMALL_ROWS:
        scratch.append(pltpu.VMEM((bpw,), jnp.int32))
        scratch.append(pltpu.VMEM((v, D), jnp.float32))

    @functools.partial(
        pl.kernel,
        mesh=mesh,
        out_type=jax.ShapeDtypeStruct((B, OUT_D), jnp.float32),
        scratch_types=scratch,
        compiler_params=pltpu.CompilerParams(needs_layout_passes=False),
    )
    def fused(g_idx, a_idx, o_idx, r_idx, w_g, w_a, w_o, w_r4, out,
              r_idx_v, r_row_v, r_buf0, r_buf1, out_v, sem0, sem1,
              g_idx_v, g_tab_v, a_idx_v, a_tab_v, o_idx_v, o_tab_v):
        wid = lax.axis_index("s") * nc + lax.axis_index("c")
        base = wid * bpw
        lane = lax.iota(jnp.int32, nl)
        nchunks = half // nl

        pltpu.sync_copy(r_idx.at[pl.ds(base, bpw)], r_idx_v)
        def rowid_body(c, _):
            idx16 = r_idx_v[pl.ds(c * nl, nl)]
            r_row_v[pl.ds(c * nl, nl)] = jnp.right_shift(idx16, 2)
            return 0
        lax.fori_loop(0, bpw // nl, rowid_body, 0)
        dma0 = pltpu.async_copy(w_r4.at[r_row_v.at[pl.ds(0, half)]], r_buf0, sem0)
        dma1 = pltpu.async_copy(w_r4.at[r_row_v.at[pl.ds(half, half)]], r_buf1, sem1)

        small = (
            (g_idx, w_g, g_idx_v, g_tab_v),
            (a_idx, w_a, a_idx_v, a_tab_v),
            (o_idx, w_o, o_idx_v, o_tab_v),
        )
        for idx_hbm, tab_hbm, idx_v, tab_v in small:
            pltpu.sync_copy(idx_hbm.at[pl.ds(base, bpw)], idx_v)
            pltpu.sync_copy(tab_hbm, tab_v)

        cols = [jnp.full((nl,), j, jnp.int32) for j in range(D)]

        def lookup_normalize(idx_v, tab_v, h, col_off, area):
            def chunk(c, _):
                row16 = c * nl + lane
                idx16 = idx_v[pl.ds(h * half + c * nl, nl)]
                if area:
                    off16 = jnp.bitwise_and(idx16, 3) * D
                vals = []
                acc = jnp.zeros((nl,), jnp.float32)
                for j in range(D):
                    if area:
                        v = plsc.load_gather(tab_v, [row16, off16 + j])
                    else:
                        v = plsc.load_gather(tab_v, [idx16, cols[j]])
                    vals.append(v)
                    acc = acc + v * v
                inv = _rsqrt_nr(jnp.maximum(acc, 1e-14))
                scale = jnp.minimum(1.0, inv)
                for j in range(D):
                    plsc.store_scatter(out_v, [row16, cols[j] + col_off],
                                       vals[j] * scale)
                return 0
            lax.fori_loop(0, nchunks, chunk, 0)

        for h, dma, r_buf in ((0, dma0, r_buf0), (1, dma1, r_buf1)):
            for t, (idx_hbm, tab_hbm, idx_v, tab_v) in enumerate(small):
                lookup_normalize(idx_v, tab_v, h, t * D, area=False)
            dma.wait()
            lookup_normalize(r_idx_v, r_buf, h, 3 * D, area=True)
            pltpu.sync_copy(out_v, out.at[pl.ds(base + h * half, half)])

    return fused


_sc_kernel = _build_sc_kernel()


def kernel(gender_idx, age_idx, occupation_idx, area_idx,
           W_gender, W_age, W_occupation, W_area):
    return _sc_kernel(
        gender_idx.astype(jnp.int32),
        age_idx.astype(jnp.int32),
        occupation_idx.astype(jnp.int32),
        area_idx.astype(jnp.int32),
        W_gender, W_age, W_occupation,
        W_area.reshape(W_area.shape[0] // AREA_FOLD, AREA_FOLD * D),
    )
